# Initial kernel scaffold; baseline (speedup 1.0000x reference)
import jax
import jax.numpy as jnp
from jax.experimental import pallas as pl


def kernel(x, edge_index, batch, W1, b1, W2, b2, Wl1, bl1, Wl2, bl2):
    raise NotImplementedError("write your pallas kernel here")


# speedup vs baseline: 7.3648x; 7.3648x over previous
import functools

import jax
import jax.numpy as jnp
from jax import lax
from jax.experimental import pallas as pl
from jax.experimental.pallas import tpu as pltpu
from jax.experimental.pallas import tpu_sc as plsc

N = 10000
D = 128
B = 64
NC, NS = 2, 16
NW = NC * NS
C = 128
NCHUNK = 80
EW = C * NCHUNK
EPAD = EW * NW
NS_ROWS = 632
NPAD = NS_ROWS * NS

BM = 2000
NBLK = N // BM

_MESH = dict(core_axis_name="c", subcore_axis_name="s",
             num_cores=NC, num_subcores=NS)


@functools.partial(
    pl.kernel,
    out_type=jax.ShapeDtypeStruct((NC, NPAD, 16), jnp.float32),
    mesh=plsc.VectorSubcoreMesh(**_MESH),
    scratch_types=[
        pltpu.VMEM((C,), jnp.int32),
        pltpu.VMEM((C, 16), jnp.float32),
        pltpu.VMEM_SHARED((NPAD, 16), jnp.float32),
    ],
)
def _sc_degree(dst_hbm, ones_hbm, zeros_hbm, out_hbm, dstv, onesv, acc):
    cid = lax.axis_index("c")
    sid = lax.axis_index("s")
    wid = sid * NC + cid
    r0 = sid * NS_ROWS
    pltpu.sync_copy(ones_hbm, onesv)
    pltpu.sync_copy(zeros_hbm.at[pl.ds(r0, NS_ROWS)], acc.at[pl.ds(r0, NS_ROWS)])
    plsc.subcore_barrier()
    base = wid * EW

    def chunk(i, carry):
        off = base + i * C
        pltpu.sync_copy(dst_hbm.at[pl.ds(off, C)], dstv)
        pltpu.sync_copy(onesv, acc.at[dstv], add=True)
        return carry

    lax.fori_loop(0, NCHUNK, chunk, 0)
    plsc.subcore_barrier()
    pltpu.sync_copy(acc.at[pl.ds(r0, NS_ROWS)], out_hbm.at[cid, pl.ds(r0, NS_ROWS)])


@functools.partial(
    pl.kernel,
    out_type=jax.ShapeDtypeStruct((NC, NPAD, D), jnp.float32),
    mesh=plsc.VectorSubcoreMesh(**_MESH),
    scratch_types=[
        pltpu.VMEM((C,), jnp.int32),
        pltpu.VMEM((C,), jnp.int32),
        pltpu.VMEM((C, D), jnp.float32),
        pltpu.VMEM_SHARED((NPAD, D), jnp.float32),
        pltpu.SemaphoreType.DMA,
    ],
)
def _sc_segsum(src_hbm, dst_hbm, table_hbm, zeros_hbm, out_hbm,
               srcv, dstv, rows, acc, sem):
    cid = lax.axis_index("c")
    sid = lax.axis_index("s")
    wid = sid * NC + cid
    r0 = sid * NS_ROWS
    pltpu.sync_copy(zeros_hbm.at[pl.ds(r0, NS_ROWS)], acc.at[pl.ds(r0, NS_ROWS)])
    plsc.subcore_barrier()
    base = wid * EW

    def chunk(i, carry):
        off = base + i * C
        pltpu.sync_copy(src_hbm.at[pl.ds(off, C)], srcv)
        pltpu.sync_copy(dst_hbm.at[pl.ds(off, C)], dstv)
        pltpu.async_copy(table_hbm.at[srcv], rows, sem).wait()
        pltpu.sync_copy(rows, acc.at[dstv], add=True)
        return carry

    lax.fori_loop(0, NCHUNK, chunk, 0)
    plsc.subcore_barrier()
    pltpu.sync_copy(acc.at[pl.ds(r0, NS_ROWS)], out_hbm.at[cid, pl.ds(r0, NS_ROWS)])


def _dinv_block(d0_ref, d1_ref):
    deg = d0_ref[:, 0:1] + d1_ref[:, 0:1] + 1.0
    return lax.rsqrt(deg)


def _tc_scale1_body(x_ref, w_ref, d0_ref, d1_ref, hp_ref):
    dinv = _dinv_block(d0_ref, d1_ref)
    h = jnp.dot(x_ref[...], w_ref[...], preferred_element_type=jnp.float32)
    hp_ref[...] = h * dinv


def _tc_mid_body(a0_ref, a1_ref, hp_ref, d0_ref, d1_ref, b1_ref, w2_ref,
                 hp2_ref):
    dinv = _dinv_block(d0_ref, d1_ref)
    t = (a0_ref[...] + a1_ref[...] + hp_ref[...]) * dinv + b1_ref[...]
    z = jnp.maximum(t, 0.0)
    hp2_ref[...] = jnp.dot(z, w2_ref[...],
                           preferred_element_type=jnp.float32) * dinv


def _tc_final_body(a0_ref, a1_ref, hp_ref, d0_ref, d1_ref, bt_ref, b2_ref,
                   wl1_ref, bl1_ref, wl2_ref, bl2_ref, out_ref, gsum, cnt):
    i = pl.program_id(0)

    @pl.when(i == 0)
    def _():
        gsum[...] = jnp.zeros_like(gsum)
        cnt[...] = jnp.zeros_like(cnt)

    dinv = _dinv_block(d0_ref, d1_ref)
    z = jnp.maximum(
        (a0_ref[...] + a1_ref[...] + hp_ref[...]) * dinv + b2_ref[...], 0.0)
    bvec = bt_ref[0]
    onehot = (jnp.broadcast_to(bvec, (B, BM)) ==
              lax.broadcasted_iota(jnp.int32, (B, BM), 0)).astype(jnp.float32)
    gsum[...] += lax.dot_general(onehot, z, (((1,), (0,)), ((), ())),
                                 preferred_element_type=jnp.float32)
    cnt[...] += jnp.broadcast_to(
        jnp.sum(onehot, axis=1, keepdims=True), (B, D))

    @pl.when(i == NBLK - 1)
    def _():
        g = gsum[...] / jnp.maximum(cnt[...], 1.0)
        gr = jnp.maximum(
            jnp.dot(g, wl1_ref[...], preferred_element_type=jnp.float32)
            + bl1_ref[...], 0.0)
        o = jnp.dot(gr, wl2_ref[...], preferred_element_type=jnp.float32)
        out_ref[...] = o[:, 0:1] + bl2_ref[...]


def _row_spec():
    return pl.BlockSpec((BM, D), lambda i: (i, 0))


def _deg_spec():
    return pl.BlockSpec((BM, 16), lambda i: (i, 0))


def _full_spec(shape):
    nd = len(shape)
    return pl.BlockSpec(shape, lambda i: (0,) * nd)


def kernel(x, edge_index, batch, W1, b1, W2, b2, Wl1, bl1, Wl2, bl2):
    f32 = jnp.float32
    src = jnp.concatenate(
        [edge_index[0], jnp.zeros((EPAD - edge_index.shape[1],), jnp.int32)])
    dst = jnp.concatenate(
        [edge_index[1],
         jnp.full((EPAD - edge_index.shape[1],), N, jnp.int32)])
    ones16 = jnp.ones((C, 16), f32)
    zeros16 = jnp.zeros((NPAD, 16), f32)
    zeros128 = jnp.zeros((NPAD, D), f32)

    degp = _sc_degree(dst, ones16, zeros16)
    d0 = degp[0, :N, :]
    d1 = degp[1, :N, :]

    hp1 = pl.pallas_call(
        _tc_scale1_body,
        grid=(NBLK,),
        in_specs=[_row_spec(), _full_spec((D, D)), _deg_spec(), _deg_spec()],
        out_specs=_row_spec(),
        out_shape=jax.ShapeDtypeStruct((N, D), f32),
    )(x, W1, d0, d1)

    acc1 = _sc_segsum(src, dst, hp1, zeros128)

    b1r = b1.reshape(1, D)
    hp2 = pl.pallas_call(
        _tc_mid_body,
        grid=(NBLK,),
        in_specs=[_row_spec(), _row_spec(), _row_spec(), _deg_spec(),
                  _deg_spec(), _full_spec((1, D)), _full_spec((D, D))],
        out_specs=_row_spec(),
        out_shape=jax.ShapeDtypeStruct((N, D), f32),
    )(acc1[0, :N], acc1[1, :N], hp1, d0, d1, b1r, W2)

    acc2 = _sc_segsum(src, dst, hp2, zeros128)

    bt = batch.reshape(NBLK, 1, BM)
    b2r = b2.reshape(1, D)
    wl2p = jnp.pad(Wl2, ((0, 0), (0, D - Wl2.shape[1])))
    bl2r = jnp.broadcast_to(bl2.reshape(1, 1), (1, 1))
    out = pl.pallas_call(
        _tc_final_body,
        grid=(NBLK,),
        in_specs=[_row_spec(), _row_spec(), _row_spec(), _deg_spec(),
                  _deg_spec(), pl.BlockSpec((1, 1, BM), lambda i: (i, 0, 0)),
                  _full_spec((1, D)), _full_spec((D, D)), _full_spec((1, D)),
                  _full_spec((D, D)), _full_spec((1, 1))],
        out_specs=pl.BlockSpec((B, 1), lambda i: (0, 0)),
        out_shape=jax.ShapeDtypeStruct((B, 1), f32),
        scratch_shapes=[pltpu.VMEM((B, D), f32), pltpu.VMEM((B, D), f32)],
    )(acc2[0, :N], acc2[1, :N], hp2, d0, d1, bt, b2r, Wl1,
      bl1.reshape(1, D), wl2p, bl2r)
    return out


# speedup vs baseline: 29.5678x; 4.0147x over previous
import functools

import jax
import jax.numpy as jnp
from jax import lax
from jax.experimental import pallas as pl
from jax.experimental.pallas import tpu as pltpu
from jax.experimental.pallas import tpu_sc as plsc

N = 10000
D = 128
B = 64
NC, NS = 2, 16
NW = NC * NS
C = 64
NCHUNK = 160
EW = C * NCHUNK
EPAD = EW * NW
NS_ROWS = 632
NPAD = NS_ROWS * NS

BM = 2000
NBLK = N // BM

_MESH = dict(core_axis_name="c", subcore_axis_name="s",
             num_cores=NC, num_subcores=NS)


NCHUNKD = NCHUNK // 2


@functools.partial(
    pl.kernel,
    out_type=jax.ShapeDtypeStruct((NC, NPAD, 16), jnp.float32),
    mesh=plsc.VectorSubcoreMesh(**_MESH),
    scratch_types=[
        pltpu.VMEM((NCHUNKD, 2 * C), jnp.int32),
        pltpu.VMEM((2 * C, 16), jnp.float32),
        pltpu.VMEM_SHARED((NPAD, 16), jnp.float32),
        pltpu.SemaphoreType.DMA,
    ],
)
def _sc_degree(dst_hbm, ones_hbm, zeros_hbm, out_hbm, dstall, onesv, acc, sem):
    cid = lax.axis_index("c")
    sid = lax.axis_index("s")
    wid = sid * NC + cid
    r0 = sid * NS_ROWS
    pltpu.sync_copy(ones_hbm, onesv)
    pltpu.sync_copy(dst_hbm.at[wid], dstall)
    pltpu.sync_copy(zeros_hbm.at[pl.ds(r0, NS_ROWS)], acc.at[pl.ds(r0, NS_ROWS)])
    plsc.subcore_barrier()

    def fire(s):
        pltpu.async_copy(onesv, acc.at[dstall.at[s]], sem, add=True)

    def wait(s):
        pltpu.make_async_copy(onesv, acc.at[dstall.at[s]], sem).wait()

    for s in range(8):
        fire(s)

    def chunk(s, carry):
        fire(s)
        wait(s - 8)
        return carry

    lax.fori_loop(8, NCHUNKD, chunk, 0)
    for s in range(NCHUNKD - 8, NCHUNKD):
        wait(s)
    plsc.subcore_barrier()
    pltpu.sync_copy(acc.at[pl.ds(r0, NS_ROWS)], out_hbm.at[cid, pl.ds(r0, NS_ROWS)])


@functools.partial(
    pl.kernel,
    out_type=jax.ShapeDtypeStruct((NC, NPAD, D), jnp.float32),
    mesh=plsc.VectorSubcoreMesh(**_MESH),
    scratch_types=[
        pltpu.VMEM((C,), jnp.int32),
        pltpu.VMEM((C,), jnp.int32),
        pltpu.VMEM((C,), jnp.int32),
        pltpu.VMEM((C,), jnp.int32),
        pltpu.VMEM((C,), jnp.int32),
        pltpu.VMEM((C,), jnp.int32),
        pltpu.VMEM((C,), jnp.int32),
        pltpu.VMEM((C,), jnp.int32),
        pltpu.VMEM((C, D), jnp.float32),
        pltpu.VMEM((C, D), jnp.float32),
        pltpu.VMEM((C, D), jnp.float32),
        pltpu.VMEM((C, D), jnp.float32),
        pltpu.VMEM_SHARED((NPAD, D), jnp.float32),
        pltpu.SemaphoreType.DMA,
        pltpu.SemaphoreType.DMA,
        pltpu.SemaphoreType.DMA,
        pltpu.SemaphoreType.DMA,
        pltpu.SemaphoreType.DMA,
        pltpu.SemaphoreType.DMA,
        pltpu.SemaphoreType.DMA,
        pltpu.SemaphoreType.DMA,
        pltpu.SemaphoreType.DMA,
        pltpu.SemaphoreType.DMA,
        pltpu.SemaphoreType.DMA,
        pltpu.SemaphoreType.DMA,
        pltpu.SemaphoreType.DMA,
        pltpu.SemaphoreType.DMA,
        pltpu.SemaphoreType.DMA,
        pltpu.SemaphoreType.DMA,
    ],
)
def _sc_segsum(src_hbm, dst_hbm, table_hbm, zeros_hbm, out_hbm,
               ib0, ib1, ib2, ib3, db0, db1, db2, db3,
               rb0, rb1, rb2, rb3, acc,
               g0, g1, g2, g3, s0, s1, s2, s3,
               i0, i1, i2, i3, d0, d1, d2, d3):
    cid = lax.axis_index("c")
    sid = lax.axis_index("s")
    wid = sid * NC + cid
    r0 = sid * NS_ROWS
    rows = [rb0, rb1, rb2, rb3]
    ibuf = [ib0, ib1, ib2, ib3]
    dbuf = [db0, db1, db2, db3]
    semg = [g0, g1, g2, g3]
    sems = [s0, s1, s2, s3]
    semi = [i0, i1, i2, i3]
    semd = [d0, d1, d2, d3]
    pltpu.sync_copy(zeros_hbm.at[pl.ds(r0, NS_ROWS)], acc.at[pl.ds(r0, NS_ROWS)])
    plsc.subcore_barrier()

    def ifire(c, r):
        pltpu.async_copy(src_hbm.at[wid, c], ibuf[r], semi[r])

    def iwait(c, r):
        pltpu.make_async_copy(src_hbm.at[wid, c], ibuf[r], semi[r]).wait()

    def dfire(c, r):
        pltpu.async_copy(dst_hbm.at[wid, c], dbuf[r], semd[r])

    def dwait(c, r):
        pltpu.make_async_copy(dst_hbm.at[wid, c], dbuf[r], semd[r]).wait()

    def gfire(c, r):
        pltpu.async_copy(table_hbm.at[ibuf[r]], rows[r], semg[r])

    def gwait(c, r):
        pltpu.make_async_copy(table_hbm.at[ibuf[r]], rows[r], semg[r]).wait()

    def sfire(c, r):
        pltpu.async_copy(rows[r], acc.at[dbuf[r]], sems[r], add=True)

    def swait(c, r):
        pltpu.make_async_copy(rows[r], acc.at[dbuf[r]], sems[r]).wait()

    def step(s, r, do_swait=True, do_ifire=True, do_dfire=True,
             do_gfire=True):
        gwait(s, r)
        dwait(s, r)
        sfire(s, r)
        if do_swait:
            swait(s - 2, (r + 2) % 4)
        if do_dfire:
            dfire(s + 2, (r + 2) % 4)
        if do_ifire:
            ifire(s + 3, (r + 3) % 4)
        if do_gfire:
            iwait(s + 2, (r + 2) % 4)
            gfire(s + 2, (r + 2) % 4)

    dfire(0, 0)
    dfire(1, 1)
    ifire(0, 0)
    ifire(1, 1)
    ifire(2, 2)
    iwait(0, 0)
    gfire(0, 0)
    iwait(1, 1)
    gfire(1, 1)
    step(0, 0, do_swait=False)
    step(1, 1, do_swait=False)

    def body(j, carry):
        for b4 in range(4):
            step(4 * j + 2 + b4, (2 + b4) % 4)
        return carry

    lax.fori_loop(0, (NCHUNK - 8) // 4, body, 0)
    for s in range(NCHUNK - 6, NCHUNK - 3):
        step(s, s % 4, do_ifire=(s + 3 < NCHUNK))
    step(NCHUNK - 3, (NCHUNK - 3) % 4, do_ifire=False)
    step(NCHUNK - 2, (NCHUNK - 2) % 4, do_ifire=False, do_dfire=False,
         do_gfire=False)
    step(NCHUNK - 1, (NCHUNK - 1) % 4, do_ifire=False, do_dfire=False,
         do_gfire=False)
    swait(NCHUNK - 2, (NCHUNK - 2) % 4)
    swait(NCHUNK - 1, (NCHUNK - 1) % 4)
    plsc.subcore_barrier()
    pltpu.sync_copy(acc.at[pl.ds(r0, NS_ROWS)], out_hbm.at[cid, pl.ds(r0, NS_ROWS)])


def _dinv_block(d0_ref, d1_ref):
    deg = d0_ref[:, 0:1] + d1_ref[:, 0:1] + 1.0
    return lax.rsqrt(deg)


def _tc_scale1_body(x_ref, w_ref, d0_ref, d1_ref, hp_ref):
    dinv = _dinv_block(d0_ref, d1_ref)
    h = jnp.dot(x_ref[...], w_ref[...], preferred_element_type=jnp.float32)
    hp_ref[...] = h * dinv


def _tc_mid_body(a0_ref, a1_ref, hp_ref, d0_ref, d1_ref, b1_ref, w2_ref,
                 hp2_ref):
    dinv = _dinv_block(d0_ref, d1_ref)
    t = (a0_ref[...] + a1_ref[...] + hp_ref[...]) * dinv + b1_ref[...]
    z = jnp.maximum(t, 0.0)
    hp2_ref[...] = jnp.dot(z, w2_ref[...],
                           preferred_element_type=jnp.float32) * dinv


def _tc_final_body(a0_ref, a1_ref, hp_ref, d0_ref, d1_ref, bt_ref, b2_ref,
                   wl1_ref, bl1_ref, wl2_ref, bl2_ref, out_ref, gsum, cnt):
    i = pl.program_id(0)

    @pl.when(i == 0)
    def _():
        gsum[...] = jnp.zeros_like(gsum)
        cnt[...] = jnp.zeros_like(cnt)

    dinv = _dinv_block(d0_ref, d1_ref)
    z = jnp.maximum(
        (a0_ref[...] + a1_ref[...] + hp_ref[...]) * dinv + b2_ref[...], 0.0)
    bvec = bt_ref[0]
    onehot = (jnp.broadcast_to(bvec, (B, BM)) ==
              lax.broadcasted_iota(jnp.int32, (B, BM), 0)).astype(jnp.float32)
    gsum[...] += lax.dot_general(onehot, z, (((1,), (0,)), ((), ())),
                                 preferred_element_type=jnp.float32)
    cnt[...] += jnp.broadcast_to(
        jnp.sum(onehot, axis=1, keepdims=True), (B, D))

    @pl.when(i == NBLK - 1)
    def _():
        g = gsum[...] / jnp.maximum(cnt[...], 1.0)
        gr = jnp.maximum(
            jnp.dot(g, wl1_ref[...], preferred_element_type=jnp.float32)
            + bl1_ref[...], 0.0)
        o = jnp.dot(gr, wl2_ref[...], preferred_element_type=jnp.float32)
        out_ref[...] = o[:, 0:1] + bl2_ref[...]


def _row_spec():
    return pl.BlockSpec((BM, D), lambda i: (i, 0))


def _deg_spec():
    return pl.BlockSpec((BM, 16), lambda i: (i, 0))


def _full_spec(shape):
    nd = len(shape)
    return pl.BlockSpec(shape, lambda i: (0,) * nd)


def kernel(x, edge_index, batch, W1, b1, W2, b2, Wl1, bl1, Wl2, bl2):
    f32 = jnp.float32
    ne = edge_index.shape[1]
    epw = ne // NW
    padw = EW - epw
    pad_src = jnp.broadcast_to(
        (jnp.arange(padw, dtype=jnp.int32) * 41) % ne % N, (NW, padw))
    pad_dst = (N + (jnp.arange(padw, dtype=jnp.int32)[None, :]
                    + 13 * jnp.arange(NW, dtype=jnp.int32)[:, None])
               % (NPAD - N)).astype(jnp.int32)
    src = jnp.concatenate(
        [edge_index[0].reshape(NW, epw), pad_src], axis=1).reshape(
            NW, NCHUNK, C)
    dst = jnp.concatenate(
        [edge_index[1].reshape(NW, epw), pad_dst], axis=1).reshape(
            NW, NCHUNK, C)
    ones16 = jnp.ones((2 * C, 16), f32)
    zeros16 = jnp.zeros((NPAD, 16), f32)
    zeros128 = jnp.zeros((NPAD, D), f32)

    degp = _sc_degree(dst.reshape(NW, NCHUNKD, 2 * C), ones16,
                      zeros16)
    d0 = degp[0, :N, :]
    d1 = degp[1, :N, :]

    hp1 = pl.pallas_call(
        _tc_scale1_body,
        grid=(NBLK,),
        in_specs=[_row_spec(), _full_spec((D, D)), _deg_spec(), _deg_spec()],
        out_specs=_row_spec(),
        out_shape=jax.ShapeDtypeStruct((N, D), f32),
    )(x, W1, d0, d1)

    acc1 = _sc_segsum(src, dst, hp1, zeros128)

    b1r = b1.reshape(1, D)
    hp2 = pl.pallas_call(
        _tc_mid_body,
        grid=(NBLK,),
        in_specs=[_row_spec(), _row_spec(), _row_spec(), _deg_spec(),
                  _deg_spec(), _full_spec((1, D)), _full_spec((D, D))],
        out_specs=_row_spec(),
        out_shape=jax.ShapeDtypeStruct((N, D), f32),
    )(acc1[0, :N], acc1[1, :N], hp1, d0, d1, b1r, W2)

    acc2 = _sc_segsum(src, dst, hp2, zeros128)

    bt = batch.reshape(NBLK, 1, BM)
    b2r = b2.reshape(1, D)
    wl2p = jnp.pad(Wl2, ((0, 0), (0, D - Wl2.shape[1])))
    bl2r = jnp.broadcast_to(bl2.reshape(1, 1), (1, 1))
    out = pl.pallas_call(
        _tc_final_body,
        grid=(NBLK,),
        in_specs=[_row_spec(), _row_spec(), _row_spec(), _deg_spec(),
                  _deg_spec(), pl.BlockSpec((1, 1, BM), lambda i: (i, 0, 0)),
                  _full_spec((1, D)), _full_spec((D, D)), _full_spec((1, D)),
                  _full_spec((D, D)), _full_spec((1, 1))],
        out_specs=pl.BlockSpec((B, 1), lambda i: (0, 0)),
        out_shape=jax.ShapeDtypeStruct((B, 1), f32),
        scratch_shapes=[pltpu.VMEM((B, D), f32), pltpu.VMEM((B, D), f32)],
    )(acc2[0, :N], acc2[1, :N], hp2, d0, d1, bt, b2r, Wl1,
      bl1.reshape(1, D), wl2p, bl2r)
    return out
